# baseline (speedup 1.0000x reference)
import functools

import jax
import jax.numpy as jnp
from jax import lax
from jax.experimental import pallas as pl
from jax.experimental.pallas import tpu as pltpu
from jax.experimental.pallas import tpu_sc as plsc

N_NODES = 10000
D_FEAT = 128
D_AUG = 136
N_EDGES = 320000
NUM_HEADS = 4
ALPHA = 0.2

CHUNK = 40
N_CHUNKS = (N_EDGES // 32) // CHUNK
ROWS_PER_TILE = N_NODES // 16
NBUF = 4
NDB = 6
STEP_LCM = 12
MAIN_BLKS = (N_CHUNKS - 2 - 8) // STEP_LCM


def _sc_aggregate(xa, ei4d):
    mesh = plsc.VectorSubcoreMesh(core_axis_name="c", subcore_axis_name="s")

    @functools.partial(
        pl.kernel,
        mesh=mesh,
        compiler_params=pltpu.CompilerParams(use_tc_tiling_on_sc=False),
        out_type=[
            jax.ShapeDtypeStruct((2, 16, ROWS_PER_TILE, D_AUG), jnp.float32),
        ],
        scratch_types=(
            [pltpu.VMEM((N_CHUNKS, CHUNK), jnp.int32)]
            + [pltpu.VMEM((CHUNK,), jnp.int32) for _ in range(NDB)]
            + [pltpu.VMEM((CHUNK, D_AUG), jnp.float32) for _ in range(NBUF)]
            + [pltpu.VMEM_SHARED((N_NODES, D_AUG), jnp.float32)]
            + [pltpu.SemaphoreType.DMA for _ in range(2 * NBUF + NDB)]
        ),
    )
    def k(x_hbm, ei_hbm, agg_out, src_idx,
          d0, d1, d2, d3, d4, d5, b0, b1, b2, b3, agg_sp, *sems):
        cid = lax.axis_index("c")
        sid = lax.axis_index("s")
        wid = cid * 16 + sid
        bufs = (b0, b1, b2, b3)
        dbufs = (d0, d1, d2, d3, d4, d5)
        gsem = sems[:NBUF]
        ssem = sems[NBUF:2 * NBUF]
        dsem = sems[2 * NBUF:]

        def start_gather(j, b):
            pltpu.async_copy(x_hbm.at[src_idx.at[j]], bufs[b], gsem[b])

        def wait_gather(j, b):
            pltpu.make_async_copy(x_hbm.at[src_idx.at[j]], bufs[b],
                                  gsem[b]).wait()

        def start_scatter(j, b, m):
            pltpu.async_copy(bufs[b], agg_sp.at[dbufs[m]], ssem[b], add=True)

        def wait_scatter(j, b, m):
            pltpu.make_async_copy(bufs[b], agg_sp.at[dbufs[m]],
                                  ssem[b]).wait()

        def start_dstload(j, m):
            pltpu.async_copy(ei_hbm.at[1, wid, j], dbufs[m], dsem[m])

        def wait_dstload(j, m):
            pltpu.make_async_copy(ei_hbm.at[1, wid, j], dbufs[m],
                                  dsem[m]).wait()

        base = sid * ROWS_PER_TILE
        zf32 = jnp.zeros((16,), jnp.float32)

        def zero_rows(i, _):
            def zero_lane(k_, __):
                b0[i, pl.ds(k_ * 16, 16)] = zf32
                return 0
            lax.fori_loop(0, D_FEAT // 16, zero_lane, 0)
            b0[i, pl.ds(D_AUG - 16, 16)] = zf32
            return 0
        lax.fori_loop(0, CHUNK, zero_rows, 0)
        for c in range(ROWS_PER_TILE // CHUNK):
            pltpu.async_copy(b0, agg_sp.at[pl.ds(base + c * CHUNK, CHUNK)],
                             gsem[0])
        rem = ROWS_PER_TILE % CHUNK
        pltpu.async_copy(b0.at[pl.ds(0, rem)],
                         agg_sp.at[pl.ds(base + ROWS_PER_TILE - rem, rem)],
                         gsem[0])
        pltpu.sync_copy(ei_hbm.at[0, wid], src_idx)
        for c in range(ROWS_PER_TILE // CHUNK):
            pltpu.make_async_copy(b0, agg_sp.at[pl.ds(base + c * CHUNK, CHUNK)],
                                  gsem[0]).wait()
        pltpu.make_async_copy(b0.at[pl.ds(0, rem)],
                              agg_sp.at[pl.ds(base + ROWS_PER_TILE - rem, rem)],
                              gsem[0]).wait()

        plsc.subcore_barrier()

        for j in range(4):
            start_dstload(j, j)
        start_gather(0, 0)
        start_gather(1, 1)
        start_gather(2, 2)

        def emit_step(j, m4, m6, has_prev_s, has_next_dst, has_next_g):
            if has_prev_s:
                wait_scatter(j - 1, (m4 + 3) % NBUF, (m6 - 1) % NDB)
            if has_next_dst:
                start_dstload(j + 4, (m6 + 4) % NDB)
            if has_next_g:
                start_gather(j + 3, (m4 + 3) % NBUF)
            wait_dstload(j, m6)
            wait_gather(j, m4)
            start_scatter(j, m4, m6)

        emit_step(0, 0, 0, False, True, True)
        emit_step(1, 1, 1, True, True, True)

        def blk_body(blk, _):
            for b in range(STEP_LCM):
                j = 2 + blk * STEP_LCM + b
                emit_step(j, (2 + b) % NBUF, (2 + b) % NDB,
                          True, True, True)
            return 0
        lax.fori_loop(0, MAIN_BLKS, blk_body, 0)

        for j in range(2 + MAIN_BLKS * STEP_LCM, N_CHUNKS):
            emit_step(j, j % NBUF, j % NDB,
                      True, j + 4 < N_CHUNKS, j + 3 < N_CHUNKS)

        wait_scatter(N_CHUNKS - 1, (N_CHUNKS - 1) % NBUF,
                     (N_CHUNKS - 1) % NDB)

        plsc.subcore_barrier()

        pltpu.sync_copy(agg_sp.at[pl.ds(base, ROWS_PER_TILE)],
                        agg_out.at[cid, sid])

    return k(xa, ei4d)


def _tc_finish_body(agg_ref, w_ref, b_ref, out_ref):
    s = agg_ref[0] + agg_ref[1]
    d = s[:, D_FEAT:D_FEAT + 1]
    inv = 1.0 / jnp.maximum(d, 1.0)
    sn = s[:, 0:D_FEAT] * inv
    y = jnp.dot(sn, w_ref[...], preferred_element_type=jnp.float32)
    y = y + b_ref[...] * (d > 0).astype(jnp.float32)
    y = jnp.where(y >= 0, y, ALPHA * y)
    out_ref[...] = jnp.broadcast_to(y[None], out_ref.shape)


def _tc_finish(aggp, W0, b0):
    B = 2000
    grid = (N_NODES // B,)
    return pl.pallas_call(
        _tc_finish_body,
        grid=grid,
        in_specs=[
            pl.BlockSpec((2, B, D_AUG), lambda i: (0, i, 0)),
            pl.BlockSpec((D_FEAT, D_FEAT), lambda i: (0, 0)),
            pl.BlockSpec((1, D_FEAT), lambda i: (0, 0)),
        ],
        out_specs=pl.BlockSpec((16, B, D_FEAT), lambda i: (0, i, 0)),
        out_shape=jax.ShapeDtypeStruct((16, N_NODES, D_FEAT), jnp.float32),
    )(aggp, W0, b0)


def kernel(x, edge_index, W0, b0):
    ei4d = edge_index.reshape(2, 32, N_CHUNKS, CHUNK)
    xa = jnp.concatenate(
        [x, jnp.ones((N_NODES, 1), jnp.float32),
         jnp.zeros((N_NODES, D_AUG - D_FEAT - 1), jnp.float32)], axis=1)
    (aggp,) = _sc_aggregate(xa, ei4d)
    aggp = aggp.reshape(2, N_NODES, D_AUG)
    out = _tc_finish(aggp, W0, b0.reshape(1, D_FEAT))
    return out.reshape(4, NUM_HEADS, N_NODES, D_FEAT)


# speedup vs baseline: 1.2551x; 1.2551x over previous
import functools

import jax
import jax.numpy as jnp
from jax import lax
from jax.experimental import pallas as pl
from jax.experimental.pallas import tpu as pltpu
from jax.experimental.pallas import tpu_sc as plsc

N_NODES = 10000
D_FEAT = 128
N_EDGES = 320000
NUM_HEADS = 4
ALPHA = 0.2

CHUNK = 40
N_CHUNKS = (N_EDGES // 32) // CHUNK
ROWS_PER_TILE = N_NODES // 16
NBUF = 4
NDB = 6
DEG_W = 8
STEP_LCM = 12
MAIN_BLKS = (N_CHUNKS - 2 - 8) // STEP_LCM


def _sc_aggregate(x, ei4d, zdeg, ones_hbm):
    mesh = plsc.VectorSubcoreMesh(core_axis_name="c", subcore_axis_name="s")

    @functools.partial(
        pl.kernel,
        mesh=mesh,
        compiler_params=pltpu.CompilerParams(use_tc_tiling_on_sc=False),
        out_type=[
            jax.ShapeDtypeStruct((2, 16, ROWS_PER_TILE, D_FEAT), jnp.float32),
            jax.ShapeDtypeStruct((2, 16, ROWS_PER_TILE, DEG_W), jnp.float32),
        ],
        scratch_types=(
            [
                pltpu.VMEM((N_CHUNKS, CHUNK), jnp.int32),
                pltpu.VMEM((CHUNK, DEG_W), jnp.float32),
            ]
            + [pltpu.VMEM((CHUNK,), jnp.int32) for _ in range(NDB)]
            + [pltpu.VMEM((CHUNK, D_FEAT), jnp.float32) for _ in range(NBUF)]
            + [
                pltpu.VMEM_SHARED((N_NODES, D_FEAT), jnp.float32),
                pltpu.VMEM_SHARED((N_NODES, DEG_W), jnp.float32),
            ]
            + [pltpu.SemaphoreType.DMA for _ in range(NBUF + NBUF + NDB + 2)]
        ),
    )
    def k(x_hbm, ei_hbm, zdeg_hbm, ones_hbm_ref,
          agg_out, deg_out, src_idx, ones_buf,
          d0, d1, d2, d3, d4, d5, b0, b1, b2, b3, agg_sp, deg_sp, *sems):
        cid = lax.axis_index("c")
        sid = lax.axis_index("s")
        wid = cid * 16 + sid
        bufs = (b0, b1, b2, b3)
        dbufs = (d0, d1, d2, d3, d4, d5)
        gsem = sems[:NBUF]
        ssem = sems[NBUF:2 * NBUF]
        dsem = sems[2 * NBUF:2 * NBUF + NDB]
        osem = sems[2 * NBUF + NDB:]

        def start_gather(j, b):
            pltpu.async_copy(x_hbm.at[src_idx.at[j]], bufs[b], gsem[b])

        def wait_gather(j, b):
            pltpu.make_async_copy(x_hbm.at[src_idx.at[j]], bufs[b],
                                  gsem[b]).wait()

        def start_scatter(j, b, m):
            pltpu.async_copy(bufs[b], agg_sp.at[dbufs[m]], ssem[b], add=True)

        def wait_scatter(j, b, m):
            pltpu.make_async_copy(bufs[b], agg_sp.at[dbufs[m]],
                                  ssem[b]).wait()

        def start_dstload(j, m):
            pltpu.async_copy(ei_hbm.at[1, wid, j], dbufs[m], dsem[m])

        def wait_dstload(j, m):
            pltpu.make_async_copy(ei_hbm.at[1, wid, j], dbufs[m],
                                  dsem[m]).wait()

        def start_ones(j, m, t):
            pltpu.async_copy(ones_buf, deg_sp.at[dbufs[m]], osem[t],
                             add=True)

        def wait_ones(j, m, t):
            pltpu.make_async_copy(ones_buf, deg_sp.at[dbufs[m]],
                                  osem[t]).wait()

        base = sid * ROWS_PER_TILE
        zf32 = jnp.zeros((16,), jnp.float32)

        def zero_rows(i, _):
            def zero_lane(k_, __):
                b0[i, pl.ds(k_ * 16, 16)] = zf32
                return 0
            return lax.fori_loop(0, D_FEAT // 16, zero_lane, 0)
        lax.fori_loop(0, CHUNK, zero_rows, 0)
        for c in range(ROWS_PER_TILE // CHUNK):
            pltpu.async_copy(b0, agg_sp.at[pl.ds(base + c * CHUNK, CHUNK)],
                             gsem[0])
        rem = ROWS_PER_TILE % CHUNK
        pltpu.async_copy(b0.at[pl.ds(0, rem)],
                         agg_sp.at[pl.ds(base + ROWS_PER_TILE - rem, rem)],
                         gsem[0])
        pltpu.sync_copy(zdeg_hbm.at[sid], deg_sp.at[pl.ds(base, ROWS_PER_TILE)])
        pltpu.sync_copy(ones_hbm_ref, ones_buf)
        pltpu.sync_copy(ei_hbm.at[0, wid], src_idx)
        for c in range(ROWS_PER_TILE // CHUNK):
            pltpu.make_async_copy(b0, agg_sp.at[pl.ds(base + c * CHUNK, CHUNK)],
                                  gsem[0]).wait()
        pltpu.make_async_copy(b0.at[pl.ds(0, rem)],
                              agg_sp.at[pl.ds(base + ROWS_PER_TILE - rem, rem)],
                              gsem[0]).wait()

        plsc.subcore_barrier()

        for j in range(4):
            start_dstload(j, j)
        start_gather(0, 0)
        start_gather(1, 1)
        start_gather(2, 2)

        def emit_step(j, m4, m6, m2, has_prev_o, has_prev_s,
                      has_next_dst, has_next_g):
            if has_prev_o:
                wait_ones(j - 2, (m6 - 2) % NDB, m2)
            if has_prev_s:
                wait_scatter(j - 1, (m4 + 3) % NBUF, (m6 - 1) % NDB)
            if has_next_dst:
                start_dstload(j + 4, (m6 + 4) % NDB)
            if has_next_g:
                start_gather(j + 3, (m4 + 3) % NBUF)
            wait_dstload(j, m6)
            wait_gather(j, m4)
            start_scatter(j, m4, m6)
            start_ones(j, m6, m2)

        emit_step(0, 0, 0, 0, False, False, True, True)
        emit_step(1, 1, 1, 1, False, True, True, True)

        def blk_body(blk, _):
            for b in range(STEP_LCM):
                j = 2 + blk * STEP_LCM + b
                emit_step(j, (2 + b) % NBUF, (2 + b) % NDB, b % 2,
                          True, True, True, True)
            return 0
        lax.fori_loop(0, MAIN_BLKS, blk_body, 0)

        for j in range(2 + MAIN_BLKS * STEP_LCM, N_CHUNKS):
            emit_step(j, j % NBUF, j % NDB, j % 2,
                      True, True, j + 4 < N_CHUNKS, j + 3 < N_CHUNKS)

        for j in (N_CHUNKS - 2, N_CHUNKS - 1):
            wait_ones(j, j % NDB, j % 2)
        wait_scatter(N_CHUNKS - 1, (N_CHUNKS - 1) % NBUF,
                     (N_CHUNKS - 1) % NDB)

        plsc.subcore_barrier()

        pltpu.sync_copy(agg_sp.at[pl.ds(base, ROWS_PER_TILE)],
                        agg_out.at[cid, sid])
        pltpu.sync_copy(deg_sp.at[pl.ds(base, ROWS_PER_TILE)],
                        deg_out.at[cid, sid])

    return k(x, ei4d, zdeg, ones_hbm)


def _tc_finish_body(agg_ref, deg_ref, w_ref, b_ref, out_ref):
    d = deg_ref[0, :, 0:1] + deg_ref[1, :, 0:1]
    inv = 1.0 / jnp.maximum(d, 1.0)
    s = (agg_ref[0] + agg_ref[1]) * inv
    y = jnp.dot(s, w_ref[...], preferred_element_type=jnp.float32)
    y = y + b_ref[...] * (d > 0).astype(jnp.float32)
    y = jnp.where(y >= 0, y, ALPHA * y)
    out_ref[...] = jnp.broadcast_to(y[None], out_ref.shape)


def _tc_finish(aggp, degp, W0, b0):
    B = 2000
    grid = (N_NODES // B,)
    return pl.pallas_call(
        _tc_finish_body,
        grid=grid,
        in_specs=[
            pl.BlockSpec((2, B, D_FEAT), lambda i: (0, i, 0)),
            pl.BlockSpec((2, B, DEG_W), lambda i: (0, i, 0)),
            pl.BlockSpec((D_FEAT, D_FEAT), lambda i: (0, 0)),
            pl.BlockSpec((1, D_FEAT), lambda i: (0, 0)),
        ],
        out_specs=pl.BlockSpec((16, B, D_FEAT), lambda i: (0, i, 0)),
        out_shape=jax.ShapeDtypeStruct((16, N_NODES, D_FEAT), jnp.float32),
    )(aggp, degp, W0, b0)


def kernel(x, edge_index, W0, b0):
    ei4d = edge_index.reshape(2, 32, N_CHUNKS, CHUNK)
    zdeg = jnp.zeros((16, ROWS_PER_TILE, DEG_W), jnp.float32)
    ones_hbm = jnp.ones((CHUNK, DEG_W), jnp.float32)
    aggp, degp = _sc_aggregate(x, ei4d, zdeg, ones_hbm)
    aggp = aggp.reshape(2, N_NODES, D_FEAT)
    degp = degp.reshape(2, N_NODES, DEG_W)
    out = _tc_finish(aggp, degp, W0, b0.reshape(1, D_FEAT))
    return out.reshape(4, NUM_HEADS, N_NODES, D_FEAT)
